# baseline (speedup 1.0000x reference)
import functools

import jax
import jax.numpy as jnp
import numpy as np
from jax import lax
from jax.experimental import pallas as pl
from jax.experimental.pallas import tpu as pltpu
from jax.experimental.pallas import tpu_sc as plsc

_VOCAB = 100000
_D = 128
_SEQ = 2048
_BATCH = 4
_NC = 2
_NS = 16
_NW = _NC * _NS
_ROWS = (_BATCH * _SEQ) // _NW


def _pe_table() -> np.ndarray:
    pos = np.arange(_SEQ, dtype=np.float32)[:, None]
    div = np.exp(np.arange(0, _D, 2, dtype=np.float32) * (-np.log(10000.0) / _D))
    pe = np.zeros((_SEQ, _D), dtype=np.float32)
    pe[:, 0::2] = np.sin(pos * div)
    pe[:, 1::2] = np.cos(pos * div)
    return pe


_PE = _pe_table()


_G = 4
_C = _ROWS // _G


_HALF = _ROWS // 2


def _sc_body(x_hbm, pe_hbm, table_hbm, out_hbm,
             idx_v, gb0, gb1, gb2, gb3, p2, p3, pe_v, acc,
             sp01, sp23, sg0, sg1, sg2, sg3,
             sa2, sa3, so0, so1, so2, so3):
    s_idx = lax.axis_index("s")
    wid = s_idx * _NC + lax.axis_index("c")
    base = wid * _ROWS
    batch = wid // (_SEQ // _ROWS)
    col = lax.rem(base, _SEQ)
    region = s_idx * _HALF
    gbufs = (gb0, gb1, gb2, gb3)
    pbufs = {2: p2, 3: p3}
    sgs = (sg0, sg1, sg2, sg3)
    sas = {2: sa2, 3: sa3}
    sos = (so0, so1, so2, so3)
    pe01 = pltpu.async_copy(pe_hbm.at[pl.ds(col, _HALF)], pe_v, sp01)
    pe23 = pltpu.async_copy(
        pe_hbm.at[pl.ds(col + _HALF, _HALF)],
        acc.at[pl.ds(region, _HALF)], sp23)
    with jax.named_scope("idx_load"):
        pltpu.sync_copy(x_hbm.at[batch, pl.ds(col, _ROWS)], idx_v)
    with jax.named_scope("gather_issue"):
        gathers = [
            pltpu.async_copy(
                table_hbm.at[idx_v.at[pl.ds(g * _C, _C)]], gbufs[g], sgs[g])
            for g in range(_G)
        ]
    with jax.named_scope("pos_setup"):
        for g in (2, 3):
            for k in range(_C // 16):
                pbufs[g][pl.ds(k * 16, 16)] = (
                    region + (g - 2) * _C + k * 16 + lax.iota(jnp.int32, 16))
    adds = {}
    with jax.named_scope("spmem_issue"):
        pe23.wait()
        for g in (2, 3):
            gathers[g].wait()
            adds[g] = pltpu.async_copy(
                gbufs[g], acc.at[pbufs[g]], sas[g], add=True)
    with jax.named_scope("alu_path"):
        pe01.wait()
        outs = []
        for g in (0, 1):
            gathers[g].wait()
            gb = gbufs[g]
            off = g * _C

            @plsc.parallel_loop(0, _C, unroll=4)
            def add_row(i, gb=gb, off=off):
                for c in range(_D // 16):
                    sl = pl.ds(c * 16, 16)
                    plsc.addupdate(gb.at[i, sl], pe_v[off + i, sl])

            outs.append(pltpu.async_copy(
                gb, out_hbm.at[pl.ds(base + off, _C)], sos[g]))
    with jax.named_scope("spmem_out"):
        for g in (2, 3):
            adds[g].wait()
            outs.append(pltpu.async_copy(
                acc.at[pl.ds(region + (g - 2) * _C, _C)],
                out_hbm.at[pl.ds(base + g * _C, _C)], sos[g]))
        for o in outs:
            o.wait()


@functools.partial(jax.jit, static_argnames=())
def _run(x2d, pe, table):
    mesh = plsc.VectorSubcoreMesh(core_axis_name="c", subcore_axis_name="s")
    f = pl.kernel(
        _sc_body,
        mesh=mesh,
        out_type=jax.ShapeDtypeStruct((_BATCH * _SEQ, _D), jnp.float32),
        scratch_types=(
            [pltpu.VMEM((_ROWS,), jnp.int32)]
            + [pltpu.VMEM((_C, _D), jnp.float32)] * _G
            + [pltpu.VMEM((_C,), jnp.int32)] * 2
            + [pltpu.VMEM((_HALF, _D), jnp.float32)]
            + [pltpu.VMEM_SHARED((_NS * _HALF, _D), jnp.float32)]
            + [pltpu.SemaphoreType.DMA] * 12
        ),
    )
    return f(x2d, pe, table)


def kernel(x, table):
    out = _run(x, _PE, table)
    return out.reshape(_BATCH, _SEQ, _D)
